# Initial kernel scaffold; baseline (speedup 1.0000x reference)
import jax
import jax.numpy as jnp
from jax.experimental import pallas as pl


def kernel(triplets, ent_embed, rel_embed, W_ent, b_ent, W_relf, b_relf, W_rel2, b_rel2, W_rel3, b_rel3, W_a, b_a, W_fc, b_fc):
    raise NotImplementedError("write your pallas kernel here")


# speedup vs baseline: 5.5169x; 5.5169x over previous
import functools

import jax
import jax.numpy as jnp
from jax import lax
from jax.experimental import pallas as pl
from jax.experimental.pallas import tpu as pltpu
from jax.experimental.pallas import tpu_sc as plsc

N = 10000
R = 100
RP2 = 104
E = 320000
D = 128
NC = 2
NS = 16
NW = NC * NS
EP = E // NW
C = 80
NCHUNK = EP // C
RT = N // NS
ZR = 640
NPAD = NS * ZR

_HIGH = jax.lax.Precision.HIGHEST


def _dot(a, b):
    return jnp.dot(a, b, precision=_HIGH, preferred_element_type=jnp.float32)


def _pre_body(ent_ref, rele_ref, went_ref, bent_ref, wrelf_ref, brelf_ref,
              wfc_ref, bfc_ref, wa_ref, ba_ref, wrel2_ref, brel2_ref,
              psrc_o, pdst_o, prel_o, ssrc_o, sdst_o, srel_o, rel2_o):
    ent = ent_ref[...]
    ep = _dot(ent, went_ref[...].T) + bent_ref[...]
    wfc = wfc_ref[...]
    ps = _dot(ep, wfc[:, :D].T)
    pd = _dot(ep, wfc[:, D:2 * D].T)
    rp = _dot(rele_ref[...], wrelf_ref[...].T) + brelf_ref[...]
    pr = _dot(rp, wfc[:, 2 * D:].T) + bfc_ref[...]
    wa = wa_ref[...]
    psrc_o[...] = ps
    pdst_o[...] = pd
    prel_o[...] = pr
    ssrc_o[...] = _dot(ps, wa.T)
    sdst_o[...] = _dot(pd, wa.T)
    srel_o[...] = _dot(pr, wa.T) + ba_ref[...]
    rel2_o[...] = _dot(rp, wrel2_ref[...].T) + brel2_ref[...]


_pre_call = pl.pallas_call(
    _pre_body,
    out_shape=(
        jax.ShapeDtypeStruct((N, D), jnp.float32),
        jax.ShapeDtypeStruct((N, D), jnp.float32),
        jax.ShapeDtypeStruct((R, D), jnp.float32),
        jax.ShapeDtypeStruct((N, D), jnp.float32),
        jax.ShapeDtypeStruct((N, D), jnp.float32),
        jax.ShapeDtypeStruct((R, D), jnp.float32),
        jax.ShapeDtypeStruct((R, D), jnp.float32),
    ),
)


def _sc1_body(src_hbm, dst_hbm, rel_hbm, pdst_hbm, prel_hbm,
              ssrc_hbm, sdst_hbm, srel_hbm,
              hacc_o, bsum_o,
              idx_s, idx_d, idx_r, rows_d, rows_r, b16,
              sbuf1,
              hacc_sh, bsum_sh, sem):
    cid = lax.axis_index("c")
    sid = lax.axis_index("s")

    def zrow(i, _):
        r = i // 8
        j = i % 8
        rows_d[r, pl.ds(j * 16, 16)] = jnp.zeros((16,), jnp.float32)
        return 0
    lax.fori_loop(0, C * 8, zrow, 0)

    def zrow16(r, _):
        b16[r, :] = jnp.zeros((16,), jnp.float32)
        return 0
    lax.fori_loop(0, C, zrow16, 0)

    def zcopy(k, _):
        pltpu.sync_copy(rows_d, hacc_sh.at[pl.ds(sid * ZR + k * C, C)])
        pltpu.sync_copy(b16, bsum_sh.at[pl.ds(sid * ZR + k * C, C)])
        return 0
    lax.fori_loop(0, ZR // C, zcopy, 0)

    plsc.subcore_barrier()

    wid = cid * NS + sid
    base0 = wid * EP

    def chunk(i, _):
        base = base0 + i * C
        pltpu.sync_copy(src_hbm.at[pl.ds(base, C)], idx_s)
        pltpu.sync_copy(dst_hbm.at[pl.ds(base, C)], idx_d)
        pltpu.sync_copy(rel_hbm.at[pl.ds(base, C)], idx_r)
        pltpu.async_copy(pdst_hbm.at[idx_d], rows_d, sem).wait()
        pltpu.async_copy(prel_hbm.at[idx_r], rows_r, sem).wait()
        pltpu.async_copy(ssrc_hbm.at[idx_s], sbuf1, sem).wait()
        pltpu.async_copy(sdst_hbm.at[idx_d], sbuf1, sem, add=True).wait()
        pltpu.async_copy(srel_hbm.at[idx_r], sbuf1, sem, add=True).wait()

        def edge(e, _):
            v = sbuf1[e, pl.ds(0, 16)]
            v = jnp.where(v >= 0.0, v, 0.01 * v)
            bb = jnp.exp(v)
            b16[e, :] = bb
            for j in range(8):
                sl = pl.ds(j * 16, 16)
                rows_d[e, sl] = bb * (rows_d[e, sl] + rows_r[e, sl])
            return 0
        lax.fori_loop(0, C, edge, 0)

        pltpu.sync_copy(rows_d, hacc_sh.at[idx_s], add=True)
        pltpu.sync_copy(b16, bsum_sh.at[idx_s], add=True)
        return 0
    lax.fori_loop(0, NCHUNK, chunk, 0)

    plsc.subcore_barrier()

    pltpu.sync_copy(hacc_sh.at[pl.ds(sid * ZR, ZR)],
                    hacc_o.at[cid, pl.ds(sid * ZR, ZR)])

    def outexp(k, _):
        off = sid * ZR + k * C
        pltpu.sync_copy(bsum_sh.at[pl.ds(off, C)], b16)

        def exp8(r, _):
            v = b16[r, :]
            for j in range(8):
                rows_d[r, pl.ds(j * 16, 16)] = v
            return 0
        lax.fori_loop(0, C, exp8, 0)
        pltpu.sync_copy(rows_d, bsum_o.at[cid, pl.ds(off, C)])
        return 0
    lax.fori_loop(0, ZR // C, outexp, 0)


_sc1_call = functools.partial(
    pl.kernel,
    out_type=(
        jax.ShapeDtypeStruct((NC, NPAD, D), jnp.float32),
        jax.ShapeDtypeStruct((NC, NPAD, D), jnp.float32),
    ),
    mesh=plsc.VectorSubcoreMesh(core_axis_name="c", subcore_axis_name="s"),
    compiler_params=pltpu.CompilerParams(use_tc_tiling_on_sc=False),
    scratch_types=[
        pltpu.VMEM((C,), jnp.int32),
        pltpu.VMEM((C,), jnp.int32),
        pltpu.VMEM((C,), jnp.int32),
        pltpu.VMEM((C, D), jnp.float32),
        pltpu.VMEM((C, D), jnp.float32),
        pltpu.VMEM((C, 16), jnp.float32),
        pltpu.VMEM((C, D), jnp.float32),
        pltpu.VMEM_SHARED((NPAD, D), jnp.float32),
        pltpu.VMEM_SHARED((NPAD, 16), jnp.float32),
        pltpu.SemaphoreType.DMA,
    ],
)(_sc1_body)


def _mid_body(hacc_ref, bsum_ref, psrc_ref, hent_o):
    hacc = hacc_ref[0] + hacc_ref[1]
    bs = bsum_ref[0, :, 0:1] + bsum_ref[1, :, 0:1]
    hent_o[...] = jnp.where(bs > 0.0, psrc_ref[...] + hacc / bs, 0.0)


_mid_call = pl.pallas_call(
    _mid_body,
    out_shape=jax.ShapeDtypeStruct((N, D), jnp.float32),
)


def _sc2_body(src_hbm, dst_hbm, rel_hbm, hent_hbm,
              s1_o, s2_o, cnt_o,
              idx_s, idx_d, idx_r, rows1, rows2, ones16, z16,
              cntbuf, cntexp,
              s1_sh, s2_sh, cnt_sh, sem):
    cid = lax.axis_index("c")
    sid = lax.axis_index("s")

    def fill1(r, _):
        ones16[r, :] = jnp.full((16,), 1.0, jnp.float32)
        z16[r, :] = jnp.zeros((16,), jnp.float32)
        return 0
    lax.fori_loop(0, C, fill1, 0)

    def z1(i, _):
        r = i // 8
        j = i % 8
        rows1[r, pl.ds(j * 16, 16)] = jnp.zeros((16,), jnp.float32)
        return 0
    lax.fori_loop(0, C * 8, z1, 0)

    @pl.when(sid == 0)
    def _():
        pltpu.sync_copy(rows1, s1_sh.at[pl.ds(0, C)])
        pltpu.sync_copy(rows1.at[pl.ds(0, RP2 - C)], s1_sh.at[pl.ds(C, RP2 - C)])

    @pl.when(sid == 1)
    def _():
        pltpu.sync_copy(rows1, s2_sh.at[pl.ds(0, C)])
        pltpu.sync_copy(rows1.at[pl.ds(0, RP2 - C)], s2_sh.at[pl.ds(C, RP2 - C)])

    @pl.when(sid == 2)
    def _():
        pltpu.sync_copy(z16, cnt_sh.at[pl.ds(0, C)])
        pltpu.sync_copy(z16.at[pl.ds(0, RP2 - C)], cnt_sh.at[pl.ds(C, RP2 - C)])

    plsc.subcore_barrier()

    wid = cid * NS + sid
    base0 = wid * EP

    def chunk(i, _):
        base = base0 + i * C
        pltpu.sync_copy(src_hbm.at[pl.ds(base, C)], idx_s)
        pltpu.sync_copy(dst_hbm.at[pl.ds(base, C)], idx_d)
        pltpu.sync_copy(rel_hbm.at[pl.ds(base, C)], idx_r)
        pltpu.async_copy(hent_hbm.at[idx_s], rows1, sem).wait()
        pltpu.async_copy(hent_hbm.at[idx_d], rows2, sem).wait()
        pltpu.sync_copy(rows1, s1_sh.at[idx_r], add=True)
        pltpu.sync_copy(rows2, s2_sh.at[idx_r], add=True)
        pltpu.sync_copy(ones16, cnt_sh.at[idx_r], add=True)
        return 0
    lax.fori_loop(0, NCHUNK, chunk, 0)

    plsc.subcore_barrier()

    @pl.when(sid == 0)
    def _():
        pltpu.sync_copy(s1_sh, s1_o.at[cid])
        pltpu.sync_copy(s2_sh, s2_o.at[cid])
        pltpu.sync_copy(cnt_sh, cntbuf)

        def exp8(r, _):
            v = cntbuf[r, :]
            for j in range(8):
                cntexp[r, pl.ds(j * 16, 16)] = v
            return 0
        lax.fori_loop(0, RP2, exp8, 0)
        pltpu.sync_copy(cntexp, cnt_o.at[cid])


_sc2_call = functools.partial(
    pl.kernel,
    out_type=(
        jax.ShapeDtypeStruct((NC, RP2, D), jnp.float32),
        jax.ShapeDtypeStruct((NC, RP2, D), jnp.float32),
        jax.ShapeDtypeStruct((NC, RP2, D), jnp.float32),
    ),
    mesh=plsc.VectorSubcoreMesh(core_axis_name="c", subcore_axis_name="s"),
    compiler_params=pltpu.CompilerParams(use_tc_tiling_on_sc=False),
    scratch_types=[
        pltpu.VMEM((C,), jnp.int32),
        pltpu.VMEM((C,), jnp.int32),
        pltpu.VMEM((C,), jnp.int32),
        pltpu.VMEM((C, D), jnp.float32),
        pltpu.VMEM((C, D), jnp.float32),
        pltpu.VMEM((C, 16), jnp.float32),
        pltpu.VMEM((C, 16), jnp.float32),
        pltpu.VMEM((RP2, 16), jnp.float32),
        pltpu.VMEM((RP2, D), jnp.float32),
        pltpu.VMEM_SHARED((RP2, D), jnp.float32),
        pltpu.VMEM_SHARED((RP2, D), jnp.float32),
        pltpu.VMEM_SHARED((RP2, 16), jnp.float32),
        pltpu.SemaphoreType.DMA,
    ],
)(_sc2_body)


def _post_body(s1_ref, s2_ref, cnt_ref, rel2_ref, wrel3_ref, brel3_ref, hrel_o):
    s1 = s1_ref[0] + s1_ref[1]
    s2 = s2_ref[0] + s2_ref[1]
    cnt = cnt_ref[0, :, 0:1] + cnt_ref[1, :, 0:1]
    w3 = wrel3_ref[...]
    hrel_o[...] = ((_dot(s1, w3[:, :D].T) + _dot(s2, w3[:, D:2 * D].T)) / cnt
                   + _dot(rel2_ref[...], w3[:, 2 * D:].T) + brel3_ref[...])


_post_call = pl.pallas_call(
    _post_body,
    out_shape=jax.ShapeDtypeStruct((R, D), jnp.float32),
)


def kernel(triplets, ent_embed, rel_embed, W_ent, b_ent, W_relf, b_relf,
           W_rel2, b_rel2, W_rel3, b_rel3, W_a, b_a, W_fc, b_fc):
    src = triplets[:, 0].astype(jnp.int32)
    dst = triplets[:, 1].astype(jnp.int32)
    rel = triplets[:, 2].astype(jnp.int32)

    psrc, pdst, prel, ssrc, sdst, srel, rel2 = _pre_call(
        ent_embed, rel_embed, W_ent, b_ent.reshape(1, D),
        W_relf, b_relf.reshape(1, D), W_fc, b_fc.reshape(1, D),
        jnp.broadcast_to(W_a, (D, D)),
        jnp.broadcast_to(b_a.reshape(1, 1), (1, D)),
        W_rel2, b_rel2.reshape(1, D))

    srel_p = jnp.pad(srel, ((0, 128 - R), (0, 0)))

    hacc, bsum = _sc1_call(src, dst, rel, pdst, prel, ssrc, sdst, srel_p)
    hent = _mid_call(hacc[:, :N], bsum[:, :N], psrc)
    s1, s2, cnt = _sc2_call(src, dst, rel, hent)
    hrel = _post_call(s1[:, :R], s2[:, :R], cnt[:, :R], rel2,
                      W_rel3, b_rel3.reshape(1, D))
    return hent, hrel


# speedup vs baseline: 6.4674x; 1.1723x over previous
import functools

import jax
import jax.numpy as jnp
from jax import lax
from jax.experimental import pallas as pl
from jax.experimental.pallas import tpu as pltpu
from jax.experimental.pallas import tpu_sc as plsc

N = 10000
R = 100
RP2 = 104
E = 320000
D = 128
NC = 2
NS = 16
NW = NC * NS
EP = E // NW
C = 80
NCHUNK = EP // C
RT = N // NS
ZR = 640
NPAD = NS * ZR

_HIGH = jax.lax.Precision.HIGHEST


def _dot(a, b):
    return jnp.dot(a, b, precision=_HIGH, preferred_element_type=jnp.float32)


def _pre_body(ent_ref, rele_ref, went_ref, bent_ref, wrelf_ref, brelf_ref,
              wfc_ref, bfc_ref, wa_ref, ba_ref, wrel2_ref, brel2_ref,
              psrc_o, pdst_o, prel_o, ssrc_o, sdst_o, srel_o, rel2_o):
    ent = ent_ref[...]
    ep = _dot(ent, went_ref[...].T) + bent_ref[...]
    wfc = wfc_ref[...]
    ps = _dot(ep, wfc[:, :D].T)
    pd = _dot(ep, wfc[:, D:2 * D].T)
    rp = _dot(rele_ref[...], wrelf_ref[...].T) + brelf_ref[...]
    pr = _dot(rp, wfc[:, 2 * D:].T) + bfc_ref[...]
    wa = wa_ref[...]
    psrc_o[...] = ps
    pdst_o[...] = pd
    prel_o[...] = pr
    ssrc_o[...] = _dot(ps, wa.T)
    sdst_o[...] = _dot(pd, wa.T)
    srel_o[...] = _dot(pr, wa.T) + ba_ref[...]
    rel2_o[...] = _dot(rp, wrel2_ref[...].T) + brel2_ref[...]


_pre_call = pl.pallas_call(
    _pre_body,
    out_shape=(
        jax.ShapeDtypeStruct((N, D), jnp.float32),
        jax.ShapeDtypeStruct((N, D), jnp.float32),
        jax.ShapeDtypeStruct((R, D), jnp.float32),
        jax.ShapeDtypeStruct((N, D), jnp.float32),
        jax.ShapeDtypeStruct((N, D), jnp.float32),
        jax.ShapeDtypeStruct((R, D), jnp.float32),
        jax.ShapeDtypeStruct((R, D), jnp.float32),
    ),
)


def _sc1_body(src_hbm, dst_hbm, rel_hbm, pdst_hbm, prel_hbm,
              ssrc_hbm, sdst_hbm, srel_hbm,
              hacc_o, bsum_o,
              idx_s, idx_d, idx_r, rows_d, rows_r, b16,
              sbuf1,
              hacc_sh, bsum_sh, sem, sem2, sem3):
    cid = lax.axis_index("c")
    sid = lax.axis_index("s")

    def zrow(i, _):
        r = i // 8
        j = i % 8
        rows_d[r, pl.ds(j * 16, 16)] = jnp.zeros((16,), jnp.float32)
        return 0
    lax.fori_loop(0, C * 8, zrow, 0)

    def zrow16(r, _):
        b16[r, :] = jnp.zeros((16,), jnp.float32)
        return 0
    lax.fori_loop(0, C, zrow16, 0)

    def zcopy(k, _):
        pltpu.sync_copy(rows_d, hacc_sh.at[pl.ds(sid * ZR + k * C, C)])
        pltpu.sync_copy(b16, bsum_sh.at[pl.ds(sid * ZR + k * C, C)])
        return 0
    lax.fori_loop(0, ZR // C, zcopy, 0)

    plsc.subcore_barrier()

    wid = cid * NS + sid
    base0 = wid * EP

    def chunk(i, _):
        base = base0 + i * C
        pltpu.sync_copy(src_hbm.at[pl.ds(base, C)], idx_s)
        pltpu.sync_copy(dst_hbm.at[pl.ds(base, C)], idx_d)
        pltpu.sync_copy(rel_hbm.at[pl.ds(base, C)], idx_r)
        cp_d = pltpu.async_copy(pdst_hbm.at[idx_d], rows_d, sem)
        cp_r = pltpu.async_copy(prel_hbm.at[idx_r], rows_r, sem2)
        pltpu.async_copy(ssrc_hbm.at[idx_s], sbuf1, sem3).wait()
        pltpu.async_copy(sdst_hbm.at[idx_d], sbuf1, sem3, add=True).wait()
        pltpu.async_copy(srel_hbm.at[idx_r], sbuf1, sem3, add=True).wait()
        cp_d.wait()
        cp_r.wait()

        def edge(e, _):
            v = sbuf1[e, pl.ds(0, 16)]
            v = jnp.where(v >= 0.0, v, 0.01 * v)
            bb = jnp.exp(v)
            b16[e, :] = bb
            for j in range(8):
                sl = pl.ds(j * 16, 16)
                rows_d[e, sl] = bb * (rows_d[e, sl] + rows_r[e, sl])
            return 0
        lax.fori_loop(0, C, edge, 0)

        pltpu.sync_copy(rows_d, hacc_sh.at[idx_s], add=True)
        pltpu.sync_copy(b16, bsum_sh.at[idx_s], add=True)
        return 0
    lax.fori_loop(0, NCHUNK, chunk, 0)

    plsc.subcore_barrier()

    pltpu.sync_copy(hacc_sh.at[pl.ds(sid * ZR, ZR)],
                    hacc_o.at[cid, pl.ds(sid * ZR, ZR)])

    def outexp(k, _):
        off = sid * ZR + k * C
        pltpu.sync_copy(bsum_sh.at[pl.ds(off, C)], b16)

        def exp8(r, _):
            v = b16[r, :]
            for j in range(8):
                rows_d[r, pl.ds(j * 16, 16)] = v
            return 0
        lax.fori_loop(0, C, exp8, 0)
        pltpu.sync_copy(rows_d, bsum_o.at[cid, pl.ds(off, C)])
        return 0
    lax.fori_loop(0, ZR // C, outexp, 0)


_sc1_call = functools.partial(
    pl.kernel,
    out_type=(
        jax.ShapeDtypeStruct((NC, NPAD, D), jnp.float32),
        jax.ShapeDtypeStruct((NC, NPAD, D), jnp.float32),
    ),
    mesh=plsc.VectorSubcoreMesh(core_axis_name="c", subcore_axis_name="s"),
    compiler_params=pltpu.CompilerParams(use_tc_tiling_on_sc=False),
    scratch_types=[
        pltpu.VMEM((C,), jnp.int32),
        pltpu.VMEM((C,), jnp.int32),
        pltpu.VMEM((C,), jnp.int32),
        pltpu.VMEM((C, D), jnp.float32),
        pltpu.VMEM((C, D), jnp.float32),
        pltpu.VMEM((C, 16), jnp.float32),
        pltpu.VMEM((C, D), jnp.float32),
        pltpu.VMEM_SHARED((NPAD, D), jnp.float32),
        pltpu.VMEM_SHARED((NPAD, 16), jnp.float32),
        pltpu.SemaphoreType.DMA,
        pltpu.SemaphoreType.DMA,
        pltpu.SemaphoreType.DMA,
    ],
)(_sc1_body)


def _mid_body(hacc_ref, bsum_ref, psrc_ref, hent_o):
    hacc = hacc_ref[0] + hacc_ref[1]
    bs = bsum_ref[0, :, 0:1] + bsum_ref[1, :, 0:1]
    hent_o[...] = jnp.where(bs > 0.0, psrc_ref[...] + hacc / bs, 0.0)


_mid_call = pl.pallas_call(
    _mid_body,
    out_shape=jax.ShapeDtypeStruct((N, D), jnp.float32),
)


def _sc2_body(src_hbm, dst_hbm, rel_hbm, hent_hbm,
              s1_o, s2_o, cnt_o,
              idx_s, idx_d, idx_r, rows1, rows2, ones16, z16,
              cntbuf, cntexp,
              s1_sh, s2_sh, cnt_sh, sem, sem2):
    cid = lax.axis_index("c")
    sid = lax.axis_index("s")

    def fill1(r, _):
        ones16[r, :] = jnp.full((16,), 1.0, jnp.float32)
        z16[r, :] = jnp.zeros((16,), jnp.float32)
        return 0
    lax.fori_loop(0, C, fill1, 0)

    def z1(i, _):
        r = i // 8
        j = i % 8
        rows1[r, pl.ds(j * 16, 16)] = jnp.zeros((16,), jnp.float32)
        return 0
    lax.fori_loop(0, C * 8, z1, 0)

    @pl.when(sid == 0)
    def _():
        pltpu.sync_copy(rows1, s1_sh.at[pl.ds(0, C)])
        pltpu.sync_copy(rows1.at[pl.ds(0, RP2 - C)], s1_sh.at[pl.ds(C, RP2 - C)])

    @pl.when(sid == 1)
    def _():
        pltpu.sync_copy(rows1, s2_sh.at[pl.ds(0, C)])
        pltpu.sync_copy(rows1.at[pl.ds(0, RP2 - C)], s2_sh.at[pl.ds(C, RP2 - C)])

    @pl.when(sid == 2)
    def _():
        pltpu.sync_copy(z16, cnt_sh.at[pl.ds(0, C)])
        pltpu.sync_copy(z16.at[pl.ds(0, RP2 - C)], cnt_sh.at[pl.ds(C, RP2 - C)])

    plsc.subcore_barrier()

    wid = cid * NS + sid
    base0 = wid * EP

    def chunk(i, _):
        base = base0 + i * C
        pltpu.sync_copy(src_hbm.at[pl.ds(base, C)], idx_s)
        pltpu.sync_copy(dst_hbm.at[pl.ds(base, C)], idx_d)
        pltpu.sync_copy(rel_hbm.at[pl.ds(base, C)], idx_r)
        cp1 = pltpu.async_copy(hent_hbm.at[idx_s], rows1, sem)
        cp2 = pltpu.async_copy(hent_hbm.at[idx_d], rows2, sem2)
        cp1.wait()
        cs1 = pltpu.async_copy(rows1, s1_sh.at[idx_r], sem, add=True)
        cp2.wait()
        cs2 = pltpu.async_copy(rows2, s2_sh.at[idx_r], sem2, add=True)
        pltpu.sync_copy(ones16, cnt_sh.at[idx_r], add=True)
        cs1.wait()
        cs2.wait()
        return 0
    lax.fori_loop(0, NCHUNK, chunk, 0)

    plsc.subcore_barrier()

    @pl.when(sid == 0)
    def _():
        pltpu.sync_copy(s1_sh, s1_o.at[cid])
        pltpu.sync_copy(s2_sh, s2_o.at[cid])
        pltpu.sync_copy(cnt_sh, cntbuf)

        def exp8(r, _):
            v = cntbuf[r, :]
            for j in range(8):
                cntexp[r, pl.ds(j * 16, 16)] = v
            return 0
        lax.fori_loop(0, RP2, exp8, 0)
        pltpu.sync_copy(cntexp, cnt_o.at[cid])


_sc2_call = functools.partial(
    pl.kernel,
    out_type=(
        jax.ShapeDtypeStruct((NC, RP2, D), jnp.float32),
        jax.ShapeDtypeStruct((NC, RP2, D), jnp.float32),
        jax.ShapeDtypeStruct((NC, RP2, D), jnp.float32),
    ),
    mesh=plsc.VectorSubcoreMesh(core_axis_name="c", subcore_axis_name="s"),
    compiler_params=pltpu.CompilerParams(use_tc_tiling_on_sc=False),
    scratch_types=[
        pltpu.VMEM((C,), jnp.int32),
        pltpu.VMEM((C,), jnp.int32),
        pltpu.VMEM((C,), jnp.int32),
        pltpu.VMEM((C, D), jnp.float32),
        pltpu.VMEM((C, D), jnp.float32),
        pltpu.VMEM((C, 16), jnp.float32),
        pltpu.VMEM((C, 16), jnp.float32),
        pltpu.VMEM((RP2, 16), jnp.float32),
        pltpu.VMEM((RP2, D), jnp.float32),
        pltpu.VMEM_SHARED((RP2, D), jnp.float32),
        pltpu.VMEM_SHARED((RP2, D), jnp.float32),
        pltpu.VMEM_SHARED((RP2, 16), jnp.float32),
        pltpu.SemaphoreType.DMA,
        pltpu.SemaphoreType.DMA,
    ],
)(_sc2_body)


def _post_body(s1_ref, s2_ref, cnt_ref, rel2_ref, wrel3_ref, brel3_ref, hrel_o):
    s1 = s1_ref[0] + s1_ref[1]
    s2 = s2_ref[0] + s2_ref[1]
    cnt = cnt_ref[0, :, 0:1] + cnt_ref[1, :, 0:1]
    w3 = wrel3_ref[...]
    hrel_o[...] = ((_dot(s1, w3[:, :D].T) + _dot(s2, w3[:, D:2 * D].T)) / cnt
                   + _dot(rel2_ref[...], w3[:, 2 * D:].T) + brel3_ref[...])


_post_call = pl.pallas_call(
    _post_body,
    out_shape=jax.ShapeDtypeStruct((R, D), jnp.float32),
)


def kernel(triplets, ent_embed, rel_embed, W_ent, b_ent, W_relf, b_relf,
           W_rel2, b_rel2, W_rel3, b_rel3, W_a, b_a, W_fc, b_fc):
    src = triplets[:, 0].astype(jnp.int32)
    dst = triplets[:, 1].astype(jnp.int32)
    rel = triplets[:, 2].astype(jnp.int32)

    psrc, pdst, prel, ssrc, sdst, srel, rel2 = _pre_call(
        ent_embed, rel_embed, W_ent, b_ent.reshape(1, D),
        W_relf, b_relf.reshape(1, D), W_fc, b_fc.reshape(1, D),
        jnp.broadcast_to(W_a, (D, D)),
        jnp.broadcast_to(b_a.reshape(1, 1), (1, D)),
        W_rel2, b_rel2.reshape(1, D))

    srel_p = jnp.pad(srel, ((0, 128 - R), (0, 0)))

    hacc, bsum = _sc1_call(src, dst, rel, pdst, prel, ssrc, sdst, srel_p)
    hent = _mid_call(hacc[:, :N], bsum[:, :N], psrc)
    s1, s2, cnt = _sc2_call(src, dst, rel, hent)
    hrel = _post_call(s1[:, :R], s2[:, :R], cnt[:, :R], rel2,
                      W_rel3, b_rel3.reshape(1, D))
    return hent, hrel
